# Initial kernel scaffold; baseline (speedup 1.0000x reference)
import jax
import jax.numpy as jnp
from jax.experimental import pallas as pl


def kernel(x, conv_w, conv_b, gamma, beta):
    raise NotImplementedError("write your pallas kernel here")


# speedup vs baseline: 2.9339x; 2.9339x over previous
import functools

import jax
import jax.numpy as jnp
from jax.experimental import pallas as pl
from jax.experimental.pallas import tpu as pltpu

_EPS = 1e-5


def _interp_mat(n_in, n_out):
    dst = jnp.arange(n_out, dtype=jnp.float32)
    src = dst * (n_in - 1) / (n_out - 1)
    lo = jnp.clip(jnp.floor(src).astype(jnp.int32), 0, n_in - 2)
    frac = src - lo.astype(jnp.float32)
    rows = jnp.arange(n_out)
    m = jnp.zeros((n_out, n_in), jnp.float32)
    m = m.at[rows, lo].add(1.0 - frac)
    return m.at[rows, lo + 1].add(frac)


def _fused_conv_kernel(x_ref, e_ref, a_ref, b_ref, y_ref, psum_ref, psq_ref,
                       *, pair, hout, cw_len):
    m = jnp.dot(x_ref[0], e_ref[...], preferred_element_type=jnp.float32)
    mb = m.astype(jnp.bfloat16)
    acc = jnp.dot(a_ref[0], mb[:, :cw_len],
                  preferred_element_type=jnp.float32)
    acc = acc + jnp.dot(a_ref[1], mb[:, cw_len:2 * cw_len],
                        preferred_element_type=jnp.float32)
    acc = acc + jnp.dot(a_ref[2], mb[:, 2 * cw_len:],
                        preferred_element_type=jnp.float32)
    y = jnp.maximum(acc + b_ref[...], 0.0)
    psum_ref[0] = jnp.sum(y, axis=0, keepdims=True)
    psq_ref[0] = jnp.sum(y * y, axis=0, keepdims=True)
    for p in range(pair):
        y_ref[p] = y[p * hout:(p + 1) * hout].astype(jnp.bfloat16)


def _bn_apply_kernel(y_ref, sc_ref, sh_ref, out_ref, *, cout, wout):
    z = y_ref[...] * sc_ref[...] + sh_ref[...]
    for co in range(cout):
        out_ref[:, co] = z[:, :, co * wout:(co + 1) * wout]


def kernel(x, conv_w, conv_b, gamma, beta):
    n, cin, h, w = x.shape
    cout = conv_w.shape[0]
    hout, wout = 2 * h, 2 * w
    cw_len = cout * wout
    pair = 2 if n % 2 == 0 else 1
    nsteps = n // pair
    f32 = jnp.float32

    uh = _interp_mat(h, hout)
    uw = _interp_mat(w, wout)
    uh_pad = jnp.pad(uh, ((1, 1), (0, 0)))
    uw_pad = jnp.pad(uw, ((1, 1), (0, 0)))
    eye_p = jnp.eye(pair, dtype=f32)
    a_ops = jnp.stack([jnp.kron(eye_p, uh_pad[dy:dy + hout]) for dy in range(3)])
    a_ops = a_ops.astype(jnp.bfloat16)
    b_ops = jnp.stack([uw_pad[dx:dx + wout].T for dx in range(3)])
    e_op = jnp.einsum('oidk,kwc->iwdoc', conv_w.astype(f32), b_ops)
    e_op = e_op.reshape(cin * w, 3 * cw_len).astype(jnp.bfloat16)
    bias_lane = jnp.repeat(conv_b.astype(f32), wout).reshape(1, cw_len)

    xw = (x.astype(jnp.bfloat16).transpose(0, 2, 1, 3)
          .reshape(nsteps, pair * h, cin * w))

    conv_fn = functools.partial(_fused_conv_kernel, pair=pair, hout=hout,
                                cw_len=cw_len)
    y_alt, psum, psq = pl.pallas_call(
        conv_fn,
        out_shape=(
            jax.ShapeDtypeStruct((n, hout, cw_len), jnp.bfloat16),
            jax.ShapeDtypeStruct((nsteps, 1, cw_len), f32),
            jax.ShapeDtypeStruct((nsteps, 1, cw_len), f32),
        ),
        grid=(nsteps,),
        in_specs=[
            pl.BlockSpec((1, pair * h, cin * w), lambda i: (i, 0, 0)),
            pl.BlockSpec((cin * w, 3 * cw_len), lambda i: (0, 0)),
            pl.BlockSpec((3, pair * hout, pair * h), lambda i: (0, 0, 0)),
            pl.BlockSpec((1, cw_len), lambda i: (0, 0)),
        ],
        out_specs=(
            pl.BlockSpec((pair, hout, cw_len), lambda i: (i, 0, 0)),
            pl.BlockSpec((1, 1, cw_len), lambda i: (i, 0, 0)),
            pl.BlockSpec((1, 1, cw_len), lambda i: (i, 0, 0)),
        ),
        compiler_params=pltpu.CompilerParams(
            dimension_semantics=("parallel",)),
    )(xw, e_op, a_ops, bias_lane)

    cnt = float(n * hout * wout)
    s = psum.sum(axis=(0, 1)).reshape(cout, wout).sum(axis=1)
    sq = psq.sum(axis=(0, 1)).reshape(cout, wout).sum(axis=1)
    mean = s / cnt
    var = sq / cnt - mean * mean
    scale = gamma.astype(f32) * jax.lax.rsqrt(var + _EPS)
    shift = beta.astype(f32) - mean * scale
    scale_lane = jnp.repeat(scale, wout).reshape(1, cw_len)
    shift_lane = jnp.repeat(shift, wout).reshape(1, cw_len)

    blk = 4 if n % 4 == 0 else 1
    bn_fn = functools.partial(_bn_apply_kernel, cout=cout, wout=wout)
    out = pl.pallas_call(
        bn_fn,
        out_shape=jax.ShapeDtypeStruct((n, cout, hout, wout), f32),
        grid=(n // blk,),
        in_specs=[
            pl.BlockSpec((blk, hout, cw_len), lambda i: (i, 0, 0)),
            pl.BlockSpec((1, cw_len), lambda i: (0, 0)),
            pl.BlockSpec((1, cw_len), lambda i: (0, 0)),
        ],
        out_specs=pl.BlockSpec((blk, cout, hout, wout), lambda i: (i, 0, 0, 0)),
        compiler_params=pltpu.CompilerParams(
            dimension_semantics=("parallel",)),
    )(y_alt, scale_lane, shift_lane)
    return out


# speedup vs baseline: 3.7394x; 1.2746x over previous
import functools

import jax
import jax.numpy as jnp
from jax.experimental import pallas as pl
from jax.experimental.pallas import tpu as pltpu

_EPS = 1e-5


def _interp_mat(n_in, n_out):
    dst = jnp.arange(n_out, dtype=jnp.float32)
    src = dst * (n_in - 1) / (n_out - 1)
    lo = jnp.clip(jnp.floor(src).astype(jnp.int32), 0, n_in - 2)
    frac = (src - lo.astype(jnp.float32))[:, None]
    cols = jnp.arange(n_in, dtype=jnp.int32)[None, :]
    lo = lo[:, None]
    return jnp.where(cols == lo, 1.0 - frac, 0.0) + \
        jnp.where(cols == lo + 1, frac, 0.0)


def _conv_kernel(x_ref, e_ref, a_ref, b_ref, y_ref, psum_ref, psq_ref,
                 *, pair, group, h, hout, cw_len):
    m = jnp.dot(x_ref[0], e_ref[...], preferred_element_type=jnp.float32)
    mb = m.astype(jnp.bfloat16)
    ps = jnp.zeros_like(psum_ref[0])
    pq = jnp.zeros_like(psq_ref[0])
    for gi in range(pair // group):
        mg = mb[gi * group * h:(gi + 1) * group * h]
        acc = jnp.dot(a_ref[0], mg[:, :cw_len],
                      preferred_element_type=jnp.float32)
        acc = acc + jnp.dot(a_ref[1], mg[:, cw_len:2 * cw_len],
                            preferred_element_type=jnp.float32)
        acc = acc + jnp.dot(a_ref[2], mg[:, 2 * cw_len:],
                            preferred_element_type=jnp.float32)
        y = jnp.maximum(acc + b_ref[...], 0.0)
        ps = ps + jnp.sum(y, axis=0, keepdims=True)
        pq = pq + jnp.sum(y * y, axis=0, keepdims=True)
        for p in range(group):
            y_ref[gi * group + p] = y[p * hout:(p + 1) * hout].astype(jnp.bfloat16)
    psum_ref[0] = ps
    psq_ref[0] = pq


def _bn_apply_kernel(y_ref, sc_ref, sh_ref, out_ref, *, cout, wout):
    z = y_ref[...] * sc_ref[...] + sh_ref[...]
    for co in range(cout):
        out_ref[:, co] = z[:, :, co * wout:(co + 1) * wout]


def _pick(n, opts):
    for o in opts:
        if n % o == 0:
            return o
    return 1


def kernel(x, conv_w, conv_b, gamma, beta):
    n, cin, h, w = x.shape
    cout = conv_w.shape[0]
    hout, wout = 2 * h, 2 * w
    cw_len = cout * wout
    pair = _pick(n, (8, 4, 2))
    group = 1
    for g in (4, 2, 1):
        if pair % g == 0 and g * h <= 256:
            group = g
            break
    nsteps = n // pair
    f32 = jnp.float32

    uh = _interp_mat(h, hout)
    uw = _interp_mat(w, wout)
    uh_pad = jnp.pad(uh, ((1, 1), (0, 0)))
    uw_pad = jnp.pad(uw, ((1, 1), (0, 0)))
    eye_g = jnp.eye(group, dtype=f32)
    a_ops = jnp.stack([jnp.kron(eye_g, uh_pad[dy:dy + hout]) for dy in range(3)])
    a_ops = a_ops.astype(jnp.bfloat16)
    b_ops = jnp.stack([uw_pad[dx:dx + wout].T for dx in range(3)])
    e_op = jnp.einsum('oidk,kwc->iwdoc', conv_w.astype(f32), b_ops)
    e_op = e_op.reshape(cin * w, 3 * cw_len).astype(jnp.bfloat16)
    bias_lane = jnp.repeat(conv_b.astype(f32), wout).reshape(1, cw_len)

    xw = (x.astype(jnp.bfloat16).transpose(0, 2, 1, 3)
          .reshape(nsteps, pair * h, cin * w))

    conv_fn = functools.partial(_conv_kernel, pair=pair, group=group, h=h,
                                hout=hout, cw_len=cw_len)
    y_alt, psum, psq = pl.pallas_call(
        conv_fn,
        out_shape=(
            jax.ShapeDtypeStruct((n, hout, cw_len), jnp.bfloat16),
            jax.ShapeDtypeStruct((nsteps, 1, cw_len), f32),
            jax.ShapeDtypeStruct((nsteps, 1, cw_len), f32),
        ),
        grid=(nsteps,),
        in_specs=[
            pl.BlockSpec((1, pair * h, cin * w), lambda i: (i, 0, 0)),
            pl.BlockSpec((cin * w, 3 * cw_len), lambda i: (0, 0)),
            pl.BlockSpec((3, group * hout, group * h), lambda i: (0, 0, 0)),
            pl.BlockSpec((1, cw_len), lambda i: (0, 0)),
        ],
        out_specs=(
            pl.BlockSpec((pair, hout, cw_len), lambda i: (i, 0, 0)),
            pl.BlockSpec((1, 1, cw_len), lambda i: (i, 0, 0)),
            pl.BlockSpec((1, 1, cw_len), lambda i: (i, 0, 0)),
        ),
        compiler_params=pltpu.CompilerParams(
            dimension_semantics=("parallel",)),
    )(xw, e_op, a_ops, bias_lane)

    cnt = float(n * hout * wout)
    tot = (jnp.stack([psum, psq]).sum(axis=(1, 2))
           .reshape(2, cout, wout).sum(axis=-1))
    mean = tot[0] / cnt
    var = tot[1] / cnt - mean * mean
    scale = gamma.astype(f32) * jax.lax.rsqrt(var + _EPS)
    shift = beta.astype(f32) - mean * scale
    scale_lane = jnp.repeat(scale, wout).reshape(1, cw_len)
    shift_lane = jnp.repeat(shift, wout).reshape(1, cw_len)

    blk = _pick(n, (8, 4, 2))
    bn_fn = functools.partial(_bn_apply_kernel, cout=cout, wout=wout)
    out = pl.pallas_call(
        bn_fn,
        out_shape=jax.ShapeDtypeStruct((n, cout, hout, wout), f32),
        grid=(n // blk,),
        in_specs=[
            pl.BlockSpec((blk, hout, cw_len), lambda i: (i, 0, 0)),
            pl.BlockSpec((1, cw_len), lambda i: (0, 0)),
            pl.BlockSpec((1, cw_len), lambda i: (0, 0)),
        ],
        out_specs=pl.BlockSpec((blk, cout, hout, wout), lambda i: (i, 0, 0, 0)),
        compiler_params=pltpu.CompilerParams(
            dimension_semantics=("parallel",)),
    )(y_alt, scale_lane, shift_lane)
    return out


# speedup vs baseline: 5.6733x; 1.5172x over previous
import functools

import jax
import jax.numpy as jnp
from jax.experimental import pallas as pl
from jax.experimental.pallas import tpu as pltpu

_EPS = 1e-5


def _interp_mat(n_in, n_out):
    dst = jnp.arange(n_out, dtype=jnp.float32)
    src = dst * (n_in - 1) / (n_out - 1)
    lo = jnp.clip(jnp.floor(src).astype(jnp.int32), 0, n_in - 2)
    frac = (src - lo.astype(jnp.float32))[:, None]
    cols = jnp.arange(n_in, dtype=jnp.int32)[None, :]
    lo = lo[:, None]
    return jnp.where(cols == lo, 1.0 - frac, 0.0) + \
        jnp.where(cols == lo + 1, frac, 0.0)


def _conv_kernel(x_ref, e_ref, a_ref, b_ref, y_ref, psum_ref, psq_ref,
                 *, pair, group, h, hout, cw_len):
    cin_w = e_ref.shape[0]
    xt = jnp.transpose(x_ref[0].astype(jnp.bfloat16), (0, 2, 1, 3))
    xw = xt.reshape(pair * h, cin_w)
    m = jnp.dot(xw, e_ref[...], preferred_element_type=jnp.float32)
    mb = m.astype(jnp.bfloat16)
    ps = jnp.zeros_like(psum_ref[0])
    pq = jnp.zeros_like(psq_ref[0])
    for gi in range(pair // group):
        mg = mb[gi * group * h:(gi + 1) * group * h]
        acc = jnp.dot(a_ref[0], mg[:, :cw_len],
                      preferred_element_type=jnp.float32)
        acc = acc + jnp.dot(a_ref[1], mg[:, cw_len:2 * cw_len],
                            preferred_element_type=jnp.float32)
        acc = acc + jnp.dot(a_ref[2], mg[:, 2 * cw_len:],
                            preferred_element_type=jnp.float32)
        y = jnp.maximum(acc + b_ref[...], 0.0)
        ps = ps + jnp.sum(y, axis=0, keepdims=True)
        pq = pq + jnp.sum(y * y, axis=0, keepdims=True)
        for p in range(group):
            y_ref[gi * group + p] = y[p * hout:(p + 1) * hout].astype(jnp.bfloat16)
    psum_ref[0] = ps
    psq_ref[0] = pq


def _bn_apply_kernel(y_ref, sc_ref, sh_ref, out_ref, *, cout, wout):
    z = y_ref[...] * sc_ref[...] + sh_ref[...]
    for co in range(cout):
        out_ref[:, co] = z[:, :, co * wout:(co + 1) * wout]


def _pick(n, opts):
    for o in opts:
        if n % o == 0:
            return o
    return 1


def kernel(x, conv_w, conv_b, gamma, beta):
    n, cin, h, w = x.shape
    cout = conv_w.shape[0]
    hout, wout = 2 * h, 2 * w
    cw_len = cout * wout
    pair = _pick(n, (8, 4, 2))
    group = 1
    for g in (4, 2, 1):
        if pair % g == 0 and g * h <= 256:
            group = g
            break
    nsteps = n // pair
    f32 = jnp.float32

    uh = _interp_mat(h, hout)
    uw = _interp_mat(w, wout)
    uh_pad = jnp.pad(uh, ((1, 1), (0, 0)))
    uw_pad = jnp.pad(uw, ((1, 1), (0, 0)))
    eye_g = jnp.eye(group, dtype=f32)
    a_ops = jnp.stack([jnp.kron(eye_g, uh_pad[dy:dy + hout]) for dy in range(3)])
    a_ops = a_ops.astype(jnp.bfloat16)
    b_ops = jnp.stack([uw_pad[dx:dx + wout].T for dx in range(3)])
    wt = conv_w.astype(f32).transpose(1, 2, 0, 3)
    e_op = sum(
        wt[:, None, :, :, dx, None] * b_ops[dx][None, :, None, None, :]
        for dx in range(3)
    )
    e_op = e_op.reshape(cin * w, 3 * cw_len).astype(jnp.bfloat16)
    bias_lane = jnp.repeat(conv_b.astype(f32), wout).reshape(1, cw_len)

    xn = x.reshape(nsteps, pair, cin, h, w)

    conv_fn = functools.partial(_conv_kernel, pair=pair, group=group, h=h,
                                hout=hout, cw_len=cw_len)
    y_alt, psum, psq = pl.pallas_call(
        conv_fn,
        out_shape=(
            jax.ShapeDtypeStruct((n, hout, cw_len), jnp.bfloat16),
            jax.ShapeDtypeStruct((nsteps, 1, cw_len), f32),
            jax.ShapeDtypeStruct((nsteps, 1, cw_len), f32),
        ),
        grid=(nsteps,),
        in_specs=[
            pl.BlockSpec((1, pair, cin, h, w), lambda i: (i, 0, 0, 0, 0)),
            pl.BlockSpec((cin * w, 3 * cw_len), lambda i: (0, 0)),
            pl.BlockSpec((3, group * hout, group * h), lambda i: (0, 0, 0)),
            pl.BlockSpec((1, cw_len), lambda i: (0, 0)),
        ],
        out_specs=(
            pl.BlockSpec((pair, hout, cw_len), lambda i: (i, 0, 0)),
            pl.BlockSpec((1, 1, cw_len), lambda i: (i, 0, 0)),
            pl.BlockSpec((1, 1, cw_len), lambda i: (i, 0, 0)),
        ),
        compiler_params=pltpu.CompilerParams(
            dimension_semantics=("parallel",)),
    )(xn, e_op, a_ops, bias_lane)

    cnt = float(n * hout * wout)
    tot = (jnp.stack([psum, psq]).sum(axis=(1, 2))
           .reshape(2, cout, wout).sum(axis=-1))
    mean = tot[0] / cnt
    var = tot[1] / cnt - mean * mean
    scale = gamma.astype(f32) * jax.lax.rsqrt(var + _EPS)
    shift = beta.astype(f32) - mean * scale
    scale_lane = jnp.repeat(scale, wout).reshape(1, cw_len)
    shift_lane = jnp.repeat(shift, wout).reshape(1, cw_len)

    blk = _pick(n, (8, 4, 2))
    bn_fn = functools.partial(_bn_apply_kernel, cout=cout, wout=wout)
    out = pl.pallas_call(
        bn_fn,
        out_shape=jax.ShapeDtypeStruct((n, cout, hout, wout), f32),
        grid=(n // blk,),
        in_specs=[
            pl.BlockSpec((blk, hout, cw_len), lambda i: (i, 0, 0)),
            pl.BlockSpec((1, cw_len), lambda i: (0, 0)),
            pl.BlockSpec((1, cw_len), lambda i: (0, 0)),
        ],
        out_specs=pl.BlockSpec((blk, cout, hout, wout), lambda i: (i, 0, 0, 0)),
        compiler_params=pltpu.CompilerParams(
            dimension_semantics=("parallel",)),
    )(y_alt, scale_lane, shift_lane)
    return out


# speedup vs baseline: 5.7888x; 1.0204x over previous
import functools

import jax
import jax.numpy as jnp
from jax.experimental import pallas as pl
from jax.experimental.pallas import tpu as pltpu

_EPS = 1e-5


def _interp_mat(n_in, n_out):
    dst = jnp.arange(n_out, dtype=jnp.float32)
    src = dst * (n_in - 1) / (n_out - 1)
    lo = jnp.clip(jnp.floor(src).astype(jnp.int32), 0, n_in - 2)
    frac = (src - lo.astype(jnp.float32))[:, None]
    cols = jnp.arange(n_in, dtype=jnp.int32)[None, :]
    lo = lo[:, None]
    return jnp.where(cols == lo, 1.0 - frac, 0.0) + \
        jnp.where(cols == lo + 1, frac, 0.0)


def _conv_kernel(x_ref, e_ref, a_ref, b_ref, y_ref, psum_ref, psq_ref,
                 *, pair, group, h, hout, cw_len):
    cin_w = e_ref.shape[0]
    xt = jnp.transpose(x_ref[0].astype(jnp.bfloat16), (0, 2, 1, 3))
    xw = xt.reshape(pair * h, cin_w)
    m = jnp.dot(xw, e_ref[...], preferred_element_type=jnp.float32)
    mb = m.astype(jnp.bfloat16)
    ps = jnp.zeros_like(psum_ref[0])
    pq = jnp.zeros_like(psq_ref[0])
    for gi in range(pair // group):
        mg = mb[gi * group * h:(gi + 1) * group * h]
        acc = jnp.dot(a_ref[0], mg[:, :cw_len],
                      preferred_element_type=jnp.float32)
        acc = acc + jnp.dot(a_ref[1], mg[:, cw_len:2 * cw_len],
                            preferred_element_type=jnp.float32)
        acc = acc + jnp.dot(a_ref[2], mg[:, 2 * cw_len:],
                            preferred_element_type=jnp.float32)
        y = jnp.maximum(acc + b_ref[...], 0.0)
        ps = ps + jnp.sum(y, axis=0, keepdims=True)
        pq = pq + jnp.sum(y * y, axis=0, keepdims=True)
        for p in range(group):
            y_ref[gi * group + p] = y[p * hout:(p + 1) * hout].astype(jnp.bfloat16)
    psum_ref[0] = ps
    psq_ref[0] = pq


def _bn_apply_kernel(y_ref, sc_ref, sh_ref, out_ref, *, cout, wout):
    z = y_ref[...] * sc_ref[...] + sh_ref[...]
    for co in range(cout):
        out_ref[:, co] = z[:, :, co * wout:(co + 1) * wout]


def _pick(n, opts):
    for o in opts:
        if n % o == 0:
            return o
    return 1


def kernel(x, conv_w, conv_b, gamma, beta):
    n, cin, h, w = x.shape
    cout = conv_w.shape[0]
    hout, wout = 2 * h, 2 * w
    cw_len = cout * wout
    pair = _pick(n, (8, 4, 2))
    group = 1
    for g in (4, 2, 1):
        if pair % g == 0 and g * h <= 256:
            group = g
            break
    nsteps = n // pair
    f32 = jnp.float32

    uh = _interp_mat(h, hout)
    uw = _interp_mat(w, wout)
    uh_pad = jnp.pad(uh, ((1, 1), (0, 0)))
    uw_pad = jnp.pad(uw, ((1, 1), (0, 0)))
    eye_g = jnp.eye(group, dtype=f32)
    a_ops = jnp.stack([jnp.kron(eye_g, uh_pad[dy:dy + hout]) for dy in range(3)])
    a_ops = a_ops.astype(jnp.bfloat16)
    b_ops = jnp.stack([uw_pad[dx:dx + wout].T for dx in range(3)])
    wt = conv_w.astype(f32).transpose(1, 2, 0, 3)
    e_op = 0.0
    for dx in range(3):
        wcol = jnp.repeat(wt[:, :, :, dx].reshape(cin, 3 * cout), wout,
                          axis=1)
        brow = jnp.tile(b_ops[dx], (1, 3 * cout))
        e_op = e_op + wcol[:, None, :] * brow[None, :, :]
    e_op = e_op.reshape(cin * w, 3 * cw_len).astype(jnp.bfloat16)
    bias_lane = jnp.repeat(conv_b.astype(f32), wout).reshape(1, cw_len)

    xn = x.reshape(nsteps, pair, cin, h, w)

    conv_fn = functools.partial(_conv_kernel, pair=pair, group=group, h=h,
                                hout=hout, cw_len=cw_len)
    y_alt, psum, psq = pl.pallas_call(
        conv_fn,
        out_shape=(
            jax.ShapeDtypeStruct((n, hout, cw_len), jnp.bfloat16),
            jax.ShapeDtypeStruct((nsteps, 1, cw_len), f32),
            jax.ShapeDtypeStruct((nsteps, 1, cw_len), f32),
        ),
        grid=(nsteps,),
        in_specs=[
            pl.BlockSpec((1, pair, cin, h, w), lambda i: (i, 0, 0, 0, 0)),
            pl.BlockSpec((cin * w, 3 * cw_len), lambda i: (0, 0)),
            pl.BlockSpec((3, group * hout, group * h), lambda i: (0, 0, 0)),
            pl.BlockSpec((1, cw_len), lambda i: (0, 0)),
        ],
        out_specs=(
            pl.BlockSpec((pair, hout, cw_len), lambda i: (i, 0, 0)),
            pl.BlockSpec((1, 1, cw_len), lambda i: (i, 0, 0)),
            pl.BlockSpec((1, 1, cw_len), lambda i: (i, 0, 0)),
        ),
        compiler_params=pltpu.CompilerParams(
            dimension_semantics=("parallel",)),
    )(xn, e_op, a_ops, bias_lane)

    cnt = float(n * hout * wout)
    tot = (jnp.stack([psum, psq]).sum(axis=(1, 2))
           .reshape(2, cout, wout).sum(axis=-1))
    mean = tot[0] / cnt
    var = tot[1] / cnt - mean * mean
    scale = gamma.astype(f32) * jax.lax.rsqrt(var + _EPS)
    shift = beta.astype(f32) - mean * scale
    scale_lane = jnp.repeat(scale, wout).reshape(1, cw_len)
    shift_lane = jnp.repeat(shift, wout).reshape(1, cw_len)

    blk = _pick(n, (8, 4, 2))
    bn_fn = functools.partial(_bn_apply_kernel, cout=cout, wout=wout)
    out = pl.pallas_call(
        bn_fn,
        out_shape=jax.ShapeDtypeStruct((n, cout, hout, wout), f32),
        grid=(n // blk,),
        in_specs=[
            pl.BlockSpec((blk, hout, cw_len), lambda i: (i, 0, 0)),
            pl.BlockSpec((1, cw_len), lambda i: (0, 0)),
            pl.BlockSpec((1, cw_len), lambda i: (0, 0)),
        ],
        out_specs=pl.BlockSpec((blk, cout, hout, wout), lambda i: (i, 0, 0, 0)),
        compiler_params=pltpu.CompilerParams(
            dimension_semantics=("parallel",)),
    )(y_alt, scale_lane, shift_lane)
    return out
